# baseline (speedup 1.0000x reference)
import functools

import jax
import jax.numpy as jnp
from jax import lax
from jax.experimental import pallas as pl
from jax.experimental.pallas import tpu as pltpu
from jax.experimental.pallas import tpu_sc as plsc

_B, _T, _D = 4, 2048, 768
_N = _B * _T
_NC, _NS = 2, 16
_NW = _NC * _NS
_POS_PER_W = _T // _NW
_CHUNK = 32
_SUB = _POS_PER_W // _CHUNK
_NCHUNK = _B * _SUB
_LANES = 16


def _make_embed_kernel():
    mesh = plsc.VectorSubcoreMesh(core_axis_name="c", subcore_axis_name="s")

    @functools.partial(
        pl.kernel,
        out_type=jax.ShapeDtypeStruct((_N, _D), jnp.float32),
        mesh=mesh,
        scratch_types=[
            pltpu.VMEM((_POS_PER_W, _D), jnp.float32),
            pltpu.VMEM((_CHUNK, _D), jnp.float32),
            pltpu.VMEM((_CHUNK, _D), jnp.float32),
            pltpu.VMEM((_CHUNK,), jnp.int32),
            pltpu.VMEM((_CHUNK,), jnp.int32),
            pltpu.SemaphoreType.DMA,
            pltpu.SemaphoreType.DMA,
            pltpu.SemaphoreType.DMA,
            pltpu.SemaphoreType.DMA,
            pltpu.SemaphoreType.DMA,
        ],
    )
    def embed(idx_hbm, tok_hbm, pos_hbm, out_hbm,
              pos_v, rows0, rows1, idx0, idx1,
              sem_pos, sem_g0, sem_g1, sem_o0, sem_o1):
        wid = lax.axis_index("s") * _NC + lax.axis_index("c")
        p0 = wid * _POS_PER_W
        rows = (rows0, rows1)
        idxv = (idx0, idx1)
        sem_g = (sem_g0, sem_g1)
        sem_o = (sem_o0, sem_o1)

        pos_cp = pltpu.async_copy(pos_hbm.at[pl.ds(p0, _POS_PER_W)],
                                  pos_v, sem_pos)

        def row_base(k):
            b, s = k // _SUB, k % _SUB
            return b * _T + p0 + s * _CHUNK

        def start(k):
            buf = k % 2
            pltpu.sync_copy(idx_hbm.at[pl.ds(row_base(k), _CHUNK)], idxv[buf])
            return pltpu.async_copy(tok_hbm.at[idxv[buf]], rows[buf],
                                    sem_g[buf])

        h_g = {0: start(0)}
        h_o = {}
        pos_cp.wait()
        for k in range(_NCHUNK):
            buf = k % 2
            if k + 1 < _NCHUNK:
                if k - 1 >= 0:
                    h_o[k - 1].wait()
                h_g[k + 1] = start(k + 1)
            h_g[k].wait()
            pos_off = (k % _SUB) * _CHUNK

            @pl.loop(0, _CHUNK)
            def _row(r):
                @pl.loop(0, _D, step=_LANES, unroll=8)
                def _col(c):
                    rows[buf][r, pl.ds(c, _LANES)] = (
                        rows[buf][r, pl.ds(c, _LANES)]
                        + pos_v[pos_off + r, pl.ds(c, _LANES)])

            h_o[k] = pltpu.async_copy(
                rows[buf], out_hbm.at[pl.ds(row_base(k), _CHUNK)], sem_o[buf])
        h_o[_NCHUNK - 2].wait()
        h_o[_NCHUNK - 1].wait()

    return embed


_embed = _make_embed_kernel()


@jax.jit
def kernel(idx, tok_table, pos_table):
    idx_flat = idx.reshape(_N).astype(jnp.int32)
    out = _embed(idx_flat, tok_table, pos_table)
    return out.reshape(_B, _T, _D)


# speedup vs baseline: 1.9231x; 1.9231x over previous
import functools

import jax
import jax.numpy as jnp
from jax import lax
from jax.experimental import pallas as pl
from jax.experimental.pallas import tpu as pltpu
from jax.experimental.pallas import tpu_sc as plsc

_B, _T, _D = 4, 2048, 768
_N = _B * _T
_NC, _NS = 2, 16
_NW = _NC * _NS
_POS_PER_W = _T // _NW
_CHUNK = 32
_SUB = _POS_PER_W // _CHUNK
_NCHUNK = _B * _SUB
_LANES = 16


def _make_embed_kernel():
    mesh = plsc.VectorSubcoreMesh(core_axis_name="c", subcore_axis_name="s")

    @functools.partial(
        pl.kernel,
        out_type=jax.ShapeDtypeStruct((_N, _D), jnp.float32),
        mesh=mesh,
        scratch_types=[
            pltpu.VMEM((_POS_PER_W, _D), jnp.float32),
            pltpu.VMEM((_CHUNK, _D), jnp.float32),
            pltpu.VMEM((_CHUNK, _D), jnp.float32),
            pltpu.VMEM((_CHUNK,), jnp.int32),
            pltpu.VMEM((_CHUNK,), jnp.int32),
            pltpu.SemaphoreType.DMA,
            pltpu.SemaphoreType.DMA,
            pltpu.SemaphoreType.DMA,
            pltpu.SemaphoreType.DMA,
            pltpu.SemaphoreType.DMA,
        ],
    )
    def embed(idx_hbm, tok_hbm, pos_hbm, out_hbm,
              pos_v, rows0, rows1, idx0, idx1,
              sem_pos, sem_g0, sem_g1, sem_o0, sem_o1):
        wid = lax.axis_index("s") * _NC + lax.axis_index("c")
        p0 = wid * _POS_PER_W
        rows = (rows0, rows1)
        idxv = (idx0, idx1)
        sem_g = (sem_g0, sem_g1)
        sem_o = (sem_o0, sem_o1)

        pos_cp = pltpu.async_copy(pos_hbm.at[pl.ds(p0, _POS_PER_W)],
                                  pos_v, sem_pos)

        def row_base(k):
            b, s = k // _SUB, k % _SUB
            return b * _T + p0 + s * _CHUNK

        def start(k):
            buf = k % 2
            pltpu.sync_copy(idx_hbm.at[pl.ds(row_base(k), _CHUNK)], idxv[buf])
            return pltpu.async_copy(tok_hbm.at[idxv[buf]], rows[buf],
                                    sem_g[buf])

        h_g = {0: start(0)}
        h_o = {}
        pos_cp.wait()
        for k in range(_NCHUNK):
            buf = k % 2
            if k + 1 < _NCHUNK:
                if k - 1 >= 0:
                    h_o[k - 1].wait()
                h_g[k + 1] = start(k + 1)
            h_g[k].wait()
            pos_off = (k % _SUB) * _CHUNK

            @pl.loop(0, _CHUNK)
            def _row(r):
                @plsc.parallel_loop(0, _D, step=_LANES, unroll=8)
                def _col(c):
                    rows[buf][r, pl.ds(c, _LANES)] = (
                        rows[buf][r, pl.ds(c, _LANES)]
                        + pos_v[pos_off + r, pl.ds(c, _LANES)])

            h_o[k] = pltpu.async_copy(
                rows[buf], out_hbm.at[pl.ds(row_base(k), _CHUNK)], sem_o[buf])
        h_o[_NCHUNK - 2].wait()
        h_o[_NCHUNK - 1].wait()

    return embed


_embed = _make_embed_kernel()


@jax.jit
def kernel(idx, tok_table, pos_table):
    idx_flat = idx.reshape(_N).astype(jnp.int32)
    out = _embed(idx_flat, tok_table, pos_table)
    return out.reshape(_B, _T, _D)


# speedup vs baseline: 2.0542x; 1.0682x over previous
import functools

import jax
import jax.numpy as jnp
from jax import lax
from jax.experimental import pallas as pl
from jax.experimental.pallas import tpu as pltpu
from jax.experimental.pallas import tpu_sc as plsc

_B, _T, _D = 4, 2048, 768
_N = _B * _T
_NC, _NS = 2, 16
_NW = _NC * _NS
_POS_PER_W = _T // _NW
_CHUNK = 32
_SUB = _POS_PER_W // _CHUNK
_NCHUNK = _B * _SUB
_ROWS_PER_W = _NCHUNK * _CHUNK
_LANES = 16
_NBUF = 3


def _make_embed_kernel():
    mesh = plsc.VectorSubcoreMesh(core_axis_name="c", subcore_axis_name="s")

    @functools.partial(
        pl.kernel,
        out_type=jax.ShapeDtypeStruct((_N, _D), jnp.float32),
        mesh=mesh,
        scratch_types=[
            pltpu.VMEM((_POS_PER_W, _D), jnp.float32),
            pltpu.VMEM((_NCHUNK, _CHUNK), jnp.int32),
            pltpu.VMEM((_CHUNK, _D), jnp.float32),
            pltpu.VMEM((_CHUNK, _D), jnp.float32),
            pltpu.VMEM((_CHUNK, _D), jnp.float32),
            pltpu.SemaphoreType.DMA,
            pltpu.SemaphoreType.DMA,
            pltpu.SemaphoreType.DMA,
            pltpu.SemaphoreType.DMA,
            pltpu.SemaphoreType.DMA,
            pltpu.SemaphoreType.DMA,
            pltpu.SemaphoreType.DMA,
            pltpu.SemaphoreType.DMA,
        ],
    )
    def embed(idx_hbm, tok_hbm, pos_hbm, out_hbm,
              pos_v, idx_v, rows0, rows1, rows2,
              sem_pos, sem_idx, sem_g0, sem_g1, sem_g2,
              sem_o0, sem_o1, sem_o2):
        wid = lax.axis_index("s") * _NC + lax.axis_index("c")
        p0 = wid * _POS_PER_W
        rows = (rows0, rows1, rows2)
        sem_g = (sem_g0, sem_g1, sem_g2)
        sem_o = (sem_o0, sem_o1, sem_o2)

        pos_cp = pltpu.async_copy(pos_hbm.at[pl.ds(p0, _POS_PER_W)],
                                  pos_v, sem_pos)

        def row_base(k):
            b, s = k // _SUB, k % _SUB
            return b * _T + p0 + s * _CHUNK

        idx_cps = [
            pltpu.async_copy(idx_hbm.at[pl.ds(row_base(k), _CHUNK)],
                             idx_v.at[k], sem_idx)
            for k in range(_NCHUNK)
        ]

        def start(k):
            buf = k % _NBUF
            return pltpu.async_copy(tok_hbm.at[idx_v.at[k]], rows[buf],
                                    sem_g[buf])

        for cp in idx_cps:
            cp.wait()
        h_g = {0: start(0), 1: start(1)}
        h_o = {}
        pos_cp.wait()
        for k in range(_NCHUNK):
            buf = k % _NBUF
            if k + 2 < _NCHUNK:
                if k - 1 >= 0:
                    h_o[k - 1].wait()
                h_g[k + 2] = start(k + 2)
            h_g[k].wait()
            pos_off = (k % _SUB) * _CHUNK

            @pl.loop(0, _CHUNK)
            def _row(r):
                @plsc.parallel_loop(0, _D, step=_LANES, unroll=8)
                def _col(c):
                    rows[buf][r, pl.ds(c, _LANES)] = (
                        rows[buf][r, pl.ds(c, _LANES)]
                        + pos_v[pos_off + r, pl.ds(c, _LANES)])

            h_o[k] = pltpu.async_copy(
                rows[buf], out_hbm.at[pl.ds(row_base(k), _CHUNK)], sem_o[buf])
        for k in (_NCHUNK - 3, _NCHUNK - 2, _NCHUNK - 1):
            h_o[k].wait()

    return embed


_embed = _make_embed_kernel()


@jax.jit
def kernel(idx, tok_table, pos_table):
    idx_flat = idx.reshape(_N).astype(jnp.int32)
    out = _embed(idx_flat, tok_table, pos_table)
    return out.reshape(_B, _T, _D)
